# Initial kernel scaffold; baseline (speedup 1.0000x reference)
import jax
import jax.numpy as jnp
from jax.experimental import pallas as pl


def kernel(h, edge_index, edge_attr, W_lin, bias, W1, b1, W2, b2):
    raise NotImplementedError("write your pallas kernel here")


# speedup vs baseline: 8.3903x; 8.3903x over previous
import functools

import jax
import jax.numpy as jnp
from jax import lax
from jax.experimental import pallas as pl
from jax.experimental.pallas import tpu as pltpu
from jax.experimental.pallas import tpu_sc as plsc

NC = 2
NS = 16
NW = NC * NS
L = 16

N_PAD = 10112
ROWS_PER_TILE = N_PAD // NS
CH = 128


def _i16(v):
    return jnp.zeros((L,), jnp.int32) + v


def _sigmoid(x):
    return 0.5 * (1.0 + jnp.tanh(0.5 * x))


def _edge_mlp_body(ea, w1, b1, w2, b2, o):
    z = lax.dot_general(ea[...], w1[...], (((1,), (1,)), ((), ())),
                        preferred_element_type=jnp.float32)
    z = z + b1[...][None, :]
    z = z * _sigmoid(z)
    t = jnp.sum(z * w2[...], axis=1, keepdims=True)
    o[...] = _sigmoid(t + b2[0])


def _lin_body(h, wl, o):
    o[...] = lax.dot_general(h[...], wl[...], (((1,), (1,)), ((), ())),
                             preferred_element_type=jnp.float32)


def _dinv_body(degs, xl, dinv, xl2):
    dsum = jnp.sum(degs[...], axis=0) + 1.0
    di = lax.rsqrt(dsum)
    dinv[...] = di[:, None]
    xl2[...] = di[: xl.shape[0], None] * xl[...]


def _final_body(p, dinv, xl, bias, o):
    d = dinv[...]
    o[...] = d * ((p[0] + p[1]) + d * xl[...]) + bias[...][None, :]


def _deg_body(col_hbm, w_hbm, out_hbm, colv, wv, deg, sem):
    c = lax.axis_index("c")
    s = lax.axis_index("s")
    wid = s * NC + c
    nchunk = col_hbm.shape[0] // NW

    def zero(i, _):
        deg[pl.ds(i * L, L)] = jnp.zeros((L,), jnp.float32)
        return 0
    lax.fori_loop(0, N_PAD // L, zero, 0)

    base = wid * nchunk
    pltpu.sync_copy(col_hbm.at[pl.ds(base, nchunk)], colv)
    pltpu.sync_copy(w_hbm.at[pl.ds(base, nchunk)], wv)

    def chunk(j, _):
        for k in range(CH // L):
            idx = colv[j, pl.ds(k * L, L)]
            val = wv[j, pl.ds(k * L, L)]
            plsc.addupdate_scatter(deg, [idx], val)
        return 0
    lax.fori_loop(0, nchunk, chunk, 0)

    pltpu.sync_copy(deg, out_hbm.at[wid])


def _agg_body(row_hbm, col_hbm, w_hbm, xl2_hbm, out_hbm,
              rowv, colv, wv, rows, zbuf, acc, sem):
    c = lax.axis_index("c")
    s = lax.axis_index("s")
    wid = s * NC + c
    nchunk = row_hbm.shape[0] // NW

    def zb(i, _):
        for k in range(128 // L):
            zbuf[i, pl.ds(k * L, L)] = jnp.zeros((L,), jnp.float32)
        return 0
    lax.fori_loop(0, zbuf.shape[0], zb, 0)

    def zc(i, _):
        pltpu.sync_copy(zbuf, acc.at[pl.ds(s * ROWS_PER_TILE + i * zbuf.shape[0],
                                           zbuf.shape[0])])
        return 0
    lax.fori_loop(0, ROWS_PER_TILE // zbuf.shape[0], zc, 0)
    plsc.subcore_barrier()

    base = wid * nchunk
    pltpu.sync_copy(row_hbm.at[pl.ds(base, nchunk)], rowv)
    pltpu.sync_copy(col_hbm.at[pl.ds(base, nchunk)], colv)
    pltpu.sync_copy(w_hbm.at[pl.ds(base, nchunk)], wv)

    def chunk(j, _):
        pltpu.async_copy(xl2_hbm.at[rowv.at[j]], rows, sem).wait()

        def edge(e, _):
            ce = plsc.load_gather(wv, [_i16(j), _i16(e)])
            for k in range(128 // L):
                sl = pl.ds(k * L, L)
                rows[e, sl] = rows[e, sl] * ce
            return 0
        lax.fori_loop(0, CH, edge, 0)

        pltpu.sync_copy(rows, acc.at[colv.at[j]], add=True)
        return 0
    lax.fori_loop(0, nchunk, chunk, 0)

    plsc.subcore_barrier()
    pltpu.sync_copy(acc.at[pl.ds(s * ROWS_PER_TILE, ROWS_PER_TILE)],
                    out_hbm.at[c, pl.ds(s * ROWS_PER_TILE, ROWS_PER_TILE)])


def _agg_body_v3(row_hbm, col_hbm, w_hbm, xa_hbm, xb_hbm, out_hbm,
                 rowv, colv, wv, b0, b1, b2, b3, zbuf, acc,
                 g0, g1, g2, g3, s0, s1, s2, s3):
    c = lax.axis_index("c")
    st = lax.axis_index("s")
    wid = st * NC + c
    nch = row_hbm.shape[0] // NW
    base = wid * nch
    hid2 = acc.shape[1]
    nv = hid2 // L

    bufs = (b0, b1, b2, b3)
    gsems = (g0, g1, g2, g3)
    ssems = (s0, s1, s2, s3)

    pltpu.sync_copy(row_hbm.at[pl.ds(base, nch)], rowv)
    pltpu.sync_copy(col_hbm.at[pl.ds(base, nch)], colv)
    pltpu.sync_copy(w_hbm.at[pl.ds(base, nch)], wv)

    def zb(i, _):
        for k in range(nv):
            zbuf[i, pl.ds(k * L, L)] = jnp.zeros((L,), jnp.float32)
        return 0
    lax.fori_loop(0, zbuf.shape[0], zb, 0)

    def zero_acc():
        def zc(i, _):
            pltpu.sync_copy(
                zbuf, acc.at[pl.ds(st * ROWS_PER_TILE + i * zbuf.shape[0],
                                   zbuf.shape[0])])
            return 0
        lax.fori_loop(0, ROWS_PER_TILE // zbuf.shape[0], zc, 0)

    zero_acc()
    plsc.subcore_barrier()

    for f, xf in enumerate((xa_hbm, xb_hbm)):
        pltpu.async_copy(xf.at[rowv.at[0]], bufs[0], gsems[0])
        pltpu.async_copy(xf.at[rowv.at[1]], bufs[1], gsems[1])

        def quad(p, _):
            for q in range(4):
                m = p * 4 + q
                buf = bufs[q]
                pltpu.make_async_copy(xf.at[rowv.at[m]], buf, gsems[q]).wait()

                @plsc.parallel_loop(0, CH, unroll=8)
                def _(t):
                    ce = jnp.zeros((L,), jnp.float32) + wv[m, t]
                    for k in range(nv):
                        sl = pl.ds(k * L, L)
                        buf[t, sl] = buf[t, sl] * ce

                pltpu.async_copy(buf, acc.at[colv.at[m]], ssems[q], add=True)
                nq = (q + 2) % 4

                @pl.when(m >= 2)
                def _():
                    pltpu.make_async_copy(bufs[nq], acc.at[colv.at[m]],
                                          ssems[nq]).wait()

                @pl.when(m + 2 < nch)
                def _():
                    pltpu.async_copy(xf.at[rowv.at[m + 2]], bufs[nq],
                                     gsems[nq])
            return 0
        lax.fori_loop(0, nch // 4, quad, 0)
        pltpu.make_async_copy(bufs[(nch - 2) % 4], acc.at[colv.at[0]],
                              ssems[(nch - 2) % 4]).wait()
        pltpu.make_async_copy(bufs[(nch - 1) % 4], acc.at[colv.at[0]],
                              ssems[(nch - 1) % 4]).wait()
        plsc.subcore_barrier()
        pltpu.sync_copy(acc.at[pl.ds(st * ROWS_PER_TILE, ROWS_PER_TILE)],
                        out_hbm.at[c, f, pl.ds(st * ROWS_PER_TILE,
                                               ROWS_PER_TILE)])
        if f == 0:
            zero_acc()
            plsc.subcore_barrier()


def kernel(h, edge_index, edge_attr, W_lin, bias, W1, b1, W2, b2):
    n, hid = h.shape
    e = edge_index.shape[1]
    half = W1.shape[0]

    be = 5000
    w = pl.pallas_call(
        _edge_mlp_body,
        grid=(e // be,),
        in_specs=[
            pl.BlockSpec((be, edge_attr.shape[1]), lambda i: (i, 0)),
            pl.BlockSpec((half, edge_attr.shape[1]), lambda i: (0, 0)),
            pl.BlockSpec((half,), lambda i: (0,)),
            pl.BlockSpec((1, half), lambda i: (0, 0)),
            pl.BlockSpec(memory_space=pltpu.MemorySpace.SMEM),
        ],
        out_specs=pl.BlockSpec((be, 1), lambda i: (i, 0)),
        out_shape=jax.ShapeDtypeStruct((e, 1), jnp.float32),
    )(edge_attr, W1, b1, W2, b2)[:, 0]

    bn = 2000
    xl = pl.pallas_call(
        _lin_body,
        grid=(n // bn,),
        in_specs=[
            pl.BlockSpec((bn, hid), lambda i: (i, 0)),
            pl.BlockSpec((hid, hid), lambda i: (0, 0)),
        ],
        out_specs=pl.BlockSpec((bn, hid), lambda i: (i, 0)),
        out_shape=jax.ShapeDtypeStruct((n, hid), jnp.float32),
    )(h, W_lin)

    quantum = NW * CH * 8
    e_pad = ((e + quantum - 1) // quantum) * quantum
    pad = e_pad - e
    row2d = jnp.concatenate(
        [edge_index[0], jnp.zeros((pad,), jnp.int32)]).reshape(-1, CH)
    col2d = jnp.concatenate(
        [edge_index[1], jnp.zeros((pad,), jnp.int32)]).reshape(-1, CH)
    w2d = jnp.concatenate([w, jnp.zeros((pad,), jnp.float32)]).reshape(-1, CH)
    nchunk = e_pad // CH

    mesh = plsc.VectorSubcoreMesh(core_axis_name="c", subcore_axis_name="s",
                                  num_cores=NC, num_subcores=NS)

    degs = pl.kernel(
        _deg_body,
        out_type=jax.ShapeDtypeStruct((NW, N_PAD), jnp.float32),
        mesh=mesh,
        compiler_params=pltpu.CompilerParams(needs_layout_passes=False),
        scratch_types=[
            pltpu.VMEM((nchunk // NW, CH), jnp.int32),
            pltpu.VMEM((nchunk // NW, CH), jnp.float32),
            pltpu.VMEM((N_PAD,), jnp.float32),
            pltpu.SemaphoreType.DMA,
        ],
    )(col2d, w2d)

    dinv, xl2 = pl.pallas_call(
        _dinv_body,
        in_specs=[
            pl.BlockSpec((NW, N_PAD), lambda: (0, 0)),
            pl.BlockSpec((n, hid), lambda: (0, 0)),
        ],
        out_specs=[
            pl.BlockSpec((N_PAD, 1), lambda: (0, 0)),
            pl.BlockSpec((n, hid), lambda: (0, 0)),
        ],
        out_shape=[
            jax.ShapeDtypeStruct((N_PAD, 1), jnp.float32),
            jax.ShapeDtypeStruct((n, hid), jnp.float32),
        ],
    )(degs, xl)

    parts = pl.kernel(
        _agg_body,
        out_type=jax.ShapeDtypeStruct((NC, N_PAD, hid), jnp.float32),
        mesh=mesh,
        compiler_params=pltpu.CompilerParams(needs_layout_passes=False),
        scratch_types=[
            pltpu.VMEM((nchunk // NW, CH), jnp.int32),
            pltpu.VMEM((nchunk // NW, CH), jnp.int32),
            pltpu.VMEM((nchunk // NW, CH), jnp.float32),
            pltpu.VMEM((CH, hid), jnp.float32),
            pltpu.VMEM((8, hid), jnp.float32),
            pltpu.VMEM_SHARED((N_PAD, hid), jnp.float32),
            pltpu.SemaphoreType.DMA,
        ],
    )(row2d, col2d, w2d, xl2)

    out = pl.pallas_call(
        _final_body,
        grid=(n // bn,),
        in_specs=[
            pl.BlockSpec((NC, bn, hid), lambda i: (0, i, 0)),
            pl.BlockSpec((bn, 1), lambda i: (i, 0)),
            pl.BlockSpec((bn, hid), lambda i: (i, 0)),
            pl.BlockSpec((hid,), lambda i: (0,)),
        ],
        out_specs=pl.BlockSpec((bn, hid), lambda i: (i, 0)),
        out_shape=jax.ShapeDtypeStruct((n, hid), jnp.float32),
    )(parts, dinv, xl, bias)

    return out
